# Initial kernel scaffold; baseline (speedup 1.0000x reference)
import jax
import jax.numpy as jnp
from jax.experimental import pallas as pl


def kernel(user_id, item_id, U, V, bi):
    raise NotImplementedError("write your pallas kernel here")


# speedup vs baseline: 28.6428x; 28.6428x over previous
import functools

import jax
import jax.numpy as jnp
from jax import lax
from jax.experimental import pallas as pl
from jax.experimental.pallas import tpu as pltpu
from jax.experimental.pallas import tpu_sc as plsc

B = 16384
S = 50
D = 16
NC = 2
NS = 16
NW = NC * NS
UB = B // NW
T = 64
NT = UB // T
CHUNK = 128
NCH = T * S // CHUNK
GW = 16 * S

@functools.cache
def _build_sc_kernel():
    mesh = plsc.VectorSubcoreMesh(core_axis_name="c", subcore_axis_name="s")
    return functools.partial(
        pl.kernel,
        mesh=mesh,
        compiler_params=pltpu.CompilerParams(
            needs_layout_passes=False, use_tc_tiling_on_sc=False),
        out_type=jax.ShapeDtypeStruct((B,), jnp.float32),
        scratch_types=[
            pltpu.VMEM((8, CHUNK), jnp.int32),
            pltpu.VMEM((UB, D), jnp.float32),
            pltpu.VMEM((NT * NCH, CHUNK), jnp.int32),
            pltpu.VMEM((T * S, D), jnp.float32),
            pltpu.VMEM((T * S,), jnp.float32),
            pltpu.VMEM((UB,), jnp.float32),
            pltpu.SemaphoreType.DMA,
            pltpu.SemaphoreType.DMA,
            pltpu.SemaphoreType.DMA,
        ],
    )(_cofiset_sc)


def _cofiset_sc(user2d, item2d, U, V, bi, out, uid_v, urow_v, idx_v,
                rows_v, bi_v, out_v, sem_u, sem_v, sem_b):
    wid = lax.axis_index("s") * NC + lax.axis_index("c")
    lanes = lax.iota(jnp.int32, 16)
    inv_s = jnp.float32(1.0 / S)

    n_uid_rows = UB // CHUNK
    pltpu.sync_copy(user2d.at[pl.ds((wid // 2) * 8, 8)], uid_v)
    uid_off = (wid % 2) * n_uid_rows
    u_copies = [
        pltpu.async_copy(U.at[uid_v.at[uid_off + r]],
                         urow_v.at[pl.ds(r * CHUNK, CHUNK)], sem_u)
        for r in range(n_uid_rows)
    ]
    for c in u_copies:
        c.wait()

    pltpu.sync_copy(item2d.at[pl.ds(wid * (NT * NCH), NT * NCH)], idx_v)

    def tile_body(t, _):
        v_copies = [
            pltpu.async_copy(V.at[idx_v.at[t * NCH + c]],
                             rows_v.at[pl.ds(c * CHUNK, CHUNK)], sem_v)
            for c in range(NCH)
        ]
        b_copies = [
            pltpu.async_copy(bi.at[idx_v.at[t * NCH + c]],
                             bi_v.at[pl.ds(c * CHUNK, CHUNK)], sem_b)
            for c in range(NCH)
        ]
        for c in v_copies:
            c.wait()
        for c in b_copies:
            c.wait()

        def group_body(g, _):
            rbase = g * GW
            bvals = [bi_v[pl.ds(rbase + r * 16, 16)] for r in range(GW // 16)]
            out_vec = jnp.zeros((16,), jnp.float32)
            for j in range(16):
                row0 = rbase + j * S
                acc = [rows_v[row0 + s] for s in range(4)]
                for s in range(4, S):
                    acc[s % 4] = acc[s % 4] + rows_v[row0 + s]
                urow = urow_v[t * T + g * 16 + j]
                tj = ((acc[0] + acc[1]) + (acc[2] + acc[3])) * urow
                lo, hi = S * j, S * j + S
                for r in range(lo // 16, (hi - 1) // 16 + 1):
                    a, b_ = max(lo, 16 * r), min(hi, 16 * (r + 1))
                    if b_ - a == 16:
                        tj = tj + bvals[r]
                    else:
                        m = (lanes >= a - 16 * r) & (lanes < b_ - 16 * r)
                        tj = tj + jnp.where(m, bvals[r], jnp.float32(0.0))
                sj = jnp.sum(tj)
                out_vec = jnp.where(lanes == j, sj, out_vec)
            out_v[pl.ds(t * T + g * 16, 16)] = out_vec * inv_s
            return 0

        lax.fori_loop(0, T // 16, group_body, 0)
        return 0

    lax.fori_loop(0, NT, tile_body, 0)
    pltpu.sync_copy(out_v, out.at[pl.ds(wid * UB, UB)])


def kernel(user_id, item_id, U, V, bi):
    user2d = user_id.astype(jnp.int32).reshape(UB * NW // CHUNK, CHUNK)
    item2d = item_id.astype(jnp.int32).reshape(B * S // CHUNK, CHUNK)
    return _build_sc_kernel()(user2d, item2d, U, V, bi)


# speedup vs baseline: 29.9387x; 1.0452x over previous
import functools

import jax
import jax.numpy as jnp
from jax import lax
from jax.experimental import pallas as pl
from jax.experimental.pallas import tpu as pltpu
from jax.experimental.pallas import tpu_sc as plsc

B = 16384
S = 50
D = 16
NC = 2
NW = 32
UB = B // NW
SP = 56
T = 16
NT = UB // T
PAIRS = NT // 2


@functools.cache
def _build_sc_kernel():
    mesh = plsc.VectorSubcoreMesh(core_axis_name="c", subcore_axis_name="s")
    return functools.partial(
        pl.kernel,
        mesh=mesh,
        compiler_params=pltpu.CompilerParams(
            needs_layout_passes=False, use_tc_tiling_on_sc=False),
        out_type=jax.ShapeDtypeStruct((B,), jnp.float32),
        scratch_types=[
            pltpu.VMEM((UB,), jnp.int32),
            pltpu.VMEM((UB, D), jnp.float32),
            pltpu.VMEM((UB, SP), jnp.int32),
            pltpu.VMEM((T * SP, D), jnp.float32),
            pltpu.VMEM((T * SP, D), jnp.float32),
            pltpu.VMEM((T, SP), jnp.float32),
            pltpu.VMEM((T, SP), jnp.float32),
            pltpu.VMEM((UB,), jnp.float32),
            pltpu.SemaphoreType.DMA,
            pltpu.SemaphoreType.DMA,
            pltpu.SemaphoreType.DMA,
        ],
    )(_cofiset_sc)


def _cofiset_sc(user_id, item_id, U, V, bi, out, uid_v, urow_v, idx_v,
                rows0, rows1, bi0, bi1, out_v, sem_u, sem0, sem1):
    wid = lax.axis_index("s") * NC + lax.axis_index("c")
    lanes = lax.iota(jnp.int32, 16)
    inv_s = jnp.float32(1.0 / S)
    ubase = wid * UB

    pltpu.sync_copy(user_id.at[pl.ds(ubase, UB)], uid_v)
    u_copies = [
        pltpu.async_copy(U.at[uid_v.at[pl.ds(k * 128, 128)]],
                         urow_v.at[pl.ds(k * 128, 128)], sem_u)
        for k in range(UB // 128)
    ]
    pltpu.sync_copy(item_id.at[pl.ds(ubase, UB)], idx_v)
    for c in u_copies:
        c.wait()

    def fire(t, rows_b, bi_b, sem):
        for j in range(T):
            r = t * T + j
            pltpu.async_copy(V.at[idx_v.at[r]],
                             rows_b.at[pl.ds(j * SP, SP)], sem)
            pltpu.async_copy(bi.at[idx_v.at[r]], bi_b.at[j], sem)

    def drain(rows_b, bi_b, sem):
        for j in range(T):
            pltpu.make_async_copy(V.at[idx_v.at[j]],
                                  rows_b.at[pl.ds(j * SP, SP)], sem).wait()
            pltpu.make_async_copy(bi.at[idx_v.at[j]], bi_b.at[j], sem).wait()

    def compute(t, rows_b, bi_b):
        out_vec = jnp.zeros((16,), jnp.float32)
        for j in range(T):
            row0 = j * SP
            acc = [rows_b[row0 + s] for s in range(4)]
            for s in range(4, S):
                acc[s % 4] = acc[s % 4] + rows_b[row0 + s]
            urow = urow_v[t * T + j]
            tj = ((acc[0] + acc[1]) + (acc[2] + acc[3])) * urow
            for k in range(3):
                tj = tj + bi_b[j, pl.ds(k * 16, 16)]
            tail = bi_b[j, pl.ds(40, 16)]
            tmask = (lanes >= 8) & (lanes < 8 + S - 48)
            tj = tj + jnp.where(tmask, tail, jnp.float32(0.0))
            sj = jnp.sum(tj)
            out_vec = jnp.where(lanes == j, sj, out_vec)
        out_v[pl.ds(t * T, T)] = out_vec * inv_s

    fire(0, rows0, bi0, sem0)

    def pair_body(p, _):
        t0 = 2 * p
        fire(t0 + 1, rows1, bi1, sem1)
        drain(rows0, bi0, sem0)
        compute(t0, rows0, bi0)

        @pl.when(p < PAIRS - 1)
        def _():
            fire(t0 + 2, rows0, bi0, sem0)

        drain(rows1, bi1, sem1)
        compute(t0 + 1, rows1, bi1)
        return 0

    lax.fori_loop(0, PAIRS, pair_body, 0)
    pltpu.sync_copy(out_v, out.at[pl.ds(wid * UB, UB)])


def kernel(user_id, item_id, U, V, bi):
    item56 = jnp.pad(item_id.astype(jnp.int32), ((0, 0), (0, SP - S)),
                     mode="wrap")
    return _build_sc_kernel()(user_id.astype(jnp.int32), item56, U, V, bi)
